# baseline (speedup 1.0000x reference)
import functools

import jax
import jax.numpy as jnp
from jax import lax
from jax.experimental import pallas as pl
from jax.experimental.pallas import tpu as pltpu
from jax.experimental.pallas import tpu_sc as plsc

NC = 2
NS = 16
NW = NC * NS
CH = 128
NBUF = 4

_f32 = jnp.float32


def _agg_call(table, src_i, dst_i, zrows, NP, W, C0, C1):
    rows_pt = NP // NS

    @functools.partial(
        pl.kernel,
        out_type=jax.ShapeDtypeStruct((NC, NP, W), _f32),
        mesh=plsc.VectorSubcoreMesh(core_axis_name="c", subcore_axis_name="s"),
        compiler_params=pltpu.CompilerParams(use_tc_tiling_on_sc=False),
        scratch_types=[
            pltpu.VMEM((C0, CH), jnp.int32),
            pltpu.VMEM((C0, CH), jnp.int32),
            pltpu.VMEM((NBUF, CH, W), _f32),
            pltpu.VMEM_SHARED((NP, W), _f32),
            pltpu.VMEM_SHARED((NP, W), _f32),
            pltpu.SemaphoreType.DMA,
            pltpu.SemaphoreType.DMA,
            pltpu.SemaphoreType.DMA,
            pltpu.SemaphoreType.DMA,
            pltpu.SemaphoreType.DMA,
            pltpu.SemaphoreType.DMA,
            pltpu.SemaphoreType.DMA,
            pltpu.SemaphoreType.DMA,
        ],
    )
    def agg(table_h, src_h, dst_h, z_h, out_h, src_v, dst_v, rows_v, acc,
            table_s, g0, g1, g2, g3, s0, s1, s2, s3):
        gsem = (g0, g1, g2, g3)
        ssem = (s0, s1, s2, s3)
        cid = lax.axis_index("c")
        sid = lax.axis_index("s")
        row0 = sid * rows_pt
        chunk0 = jnp.where(cid == 0, sid * C0, NS * C0 + sid * C1)
        nch = jnp.where(cid == 0, C0, C1)
        pltpu.sync_copy(z_h, acc.at[pl.ds(row0, rows_pt)])
        pltpu.sync_copy(table_h.at[pl.ds(row0, rows_pt)],
                        table_s.at[pl.ds(row0, rows_pt)])
        pltpu.sync_copy(src_h.at[pl.ds(chunk0, C0)], src_v)
        pltpu.sync_copy(dst_h.at[pl.ds(chunk0, C0)], dst_v)
        plsc.subcore_barrier()
        for b in range(2):
            pltpu.async_copy(table_s.at[src_v.at[b]], rows_v.at[b], gsem[b])

        def step(g, carry):
            for b in range(NBUF):
                c = g * NBUF + b
                pltpu.make_async_copy(
                    table_s.at[src_v.at[c]], rows_v.at[b], gsem[b]).wait()
                pltpu.async_copy(rows_v.at[b], acc.at[dst_v.at[c]], ssem[b],
                                 add=True)
                j = c + 2
                bj = (b + 2) % NBUF

                @pl.when(j < nch)
                def _fire():
                    @pl.when(j >= NBUF)
                    def _drain():
                        pltpu.make_async_copy(
                            rows_v.at[bj], acc.at[dst_v.at[j - NBUF]],
                            ssem[bj]).wait()

                    pltpu.async_copy(
                        table_s.at[src_v.at[j]], rows_v.at[bj], gsem[bj])
            return carry

        lax.fori_loop(0, nch // NBUF, step, 0)
        for b in range(NBUF):
            c = nch - NBUF + b
            pltpu.make_async_copy(
                rows_v.at[b], acc.at[dst_v.at[c]], ssem[b]).wait()
        plsc.subcore_barrier()
        pltpu.sync_copy(acc.at[pl.ds(row0, rows_pt)],
                        out_h.at[cid, pl.ds(row0, rows_pt)])

    return agg(table, src_i, dst_i, zrows)


def _count_call(dst_i, zrows, ones_rows, NP, C0, C1):
    rows_pt = NP // NS

    @functools.partial(
        pl.kernel,
        out_type=jax.ShapeDtypeStruct((NC, NP, 8), _f32),
        mesh=plsc.VectorSubcoreMesh(core_axis_name="c", subcore_axis_name="s"),
        compiler_params=pltpu.CompilerParams(use_tc_tiling_on_sc=False),
        scratch_types=[
            pltpu.VMEM((C0, CH), jnp.int32),
            pltpu.VMEM((CH, 8), _f32),
            pltpu.VMEM_SHARED((NP, 8), _f32),
            pltpu.SemaphoreType.DMA,
            pltpu.SemaphoreType.DMA,
            pltpu.SemaphoreType.DMA,
            pltpu.SemaphoreType.DMA,
        ],
    )
    def cnt(dst_h, z_h, ones_h, out_h, dst_v, ones_v, acc, s0, s1, s2, s3):
        ssem = (s0, s1, s2, s3)
        cid = lax.axis_index("c")
        sid = lax.axis_index("s")
        row0 = sid * rows_pt
        chunk0 = jnp.where(cid == 0, sid * C0, NS * C0 + sid * C1)
        nch = jnp.where(cid == 0, C0, C1)
        pltpu.sync_copy(z_h, acc.at[pl.ds(row0, rows_pt)])
        pltpu.sync_copy(ones_h, ones_v)
        pltpu.sync_copy(dst_h.at[pl.ds(chunk0, C0)], dst_v)
        plsc.subcore_barrier()

        def step(g, carry):
            for b in range(NBUF):
                c = g * NBUF + b

                @pl.when(c >= NBUF)
                def _drain():
                    pltpu.make_async_copy(
                        ones_v, acc.at[dst_v.at[c - NBUF]], ssem[b]).wait()

                pltpu.async_copy(ones_v, acc.at[dst_v.at[c]], ssem[b],
                                 add=True)
            return carry

        lax.fori_loop(0, nch // NBUF, step, 0)
        for b in range(NBUF):
            c = nch - NBUF + b
            pltpu.make_async_copy(ones_v, acc.at[dst_v.at[c]], ssem[b]).wait()
        plsc.subcore_barrier()
        pltpu.sync_copy(acc.at[pl.ds(row0, rows_pt)],
                        out_h.at[cid, pl.ds(row0, rows_pt)])

    return cnt(dst_i, zrows, ones_rows)


_TCG = 8


def _rows(bn, w):
    return pl.BlockSpec((bn, w), lambda i: (i, 0))


def _full(s0, s1):
    return pl.BlockSpec((s0, s1), lambda i: (0, 0))


def _tc0_call(x_pad, W1, NP, D, H):
    bn = NP // _TCG

    def body(xr, w1, p_o):
        p_o[...] = jnp.dot(xr[...], w1[...], preferred_element_type=_f32)

    return pl.pallas_call(
        body,
        grid=(_TCG,),
        in_specs=[_rows(bn, D), _full(D, H)],
        out_specs=_rows(bn, H),
        out_shape=jax.ShapeDtypeStruct((NP, H), _f32),
    )(x_pad, W1)


def _tc1_call(cnt_a, cnt_b, p1, NP, H):
    bn = NP // _TCG

    def body(ca, cb, p, dinv_o, v1_o):
        dinv = lax.rsqrt(ca[...] + cb[...] + 1.0)
        dinv_o[...] = dinv
        v1_o[...] = p[...] * dinv

    return pl.pallas_call(
        body,
        grid=(_TCG,),
        in_specs=[_rows(bn, 1), _rows(bn, 1), _rows(bn, H)],
        out_specs=(_rows(bn, 1), _rows(bn, H)),
        out_shape=(jax.ShapeDtypeStruct((NP, 1), _f32),
                   jax.ShapeDtypeStruct((NP, H), _f32)),
    )(cnt_a, cnt_b, p1)


def _tc2_call(dinv, u1a, u1b, v1, W2, b1, NP, H):
    bn = NP // _TCG

    def body(di, ua, ub, v, w2, b, v2_o):
        h = di[...] * (ua[...] + ub[...] + v[...]) + b[...]
        v2_o[...] = di[...] * jnp.dot(h, w2[...], preferred_element_type=_f32)

    return pl.pallas_call(
        body,
        grid=(_TCG,),
        in_specs=[_rows(bn, 1), _rows(bn, H), _rows(bn, H), _rows(bn, H),
                  _full(H, H), _full(1, H)],
        out_specs=_rows(bn, H),
        out_shape=jax.ShapeDtypeStruct((NP, H), _f32),
    )(dinv, u1a, u1b, v1, W2, b1)


def _tc3_call(dinv, u2a, u2b, v2, w3p, w4p, b2, NP, H):
    bn = NP // _TCG

    def body(di, ua, ub, v, w3, w4, b, v3_o):
        h2 = di[...] * (ua[...] + ub[...] + v[...]) + b[...]
        r = (jnp.dot(jnp.maximum(h2, 0.0), w3[...], preferred_element_type=_f32)
             + jnp.dot(h2, w4[...], preferred_element_type=_f32))
        v3_o[...] = di[...] * r

    return pl.pallas_call(
        body,
        grid=(_TCG,),
        in_specs=[_rows(bn, 1), _rows(bn, H), _rows(bn, H), _rows(bn, H),
                  _full(H, 8), _full(H, 8), _full(1, H)],
        out_specs=_rows(bn, 8),
        out_shape=jax.ShapeDtypeStruct((NP, 8), _f32),
    )(dinv, u2a, u2b, v2, w3p, w4p, b2)


def _tc4_call(dinv, u3a, u3b, v3, b34, NP):
    bn = NP // _TCG

    def body(di, ua, ub, v, b, mu_o, std_o):
        od = di[...] * (ua[...] + ub[...] + v[...]) + b[...]
        mu_o[...] = od[:, 0:1]
        std_o[...] = od[:, 1:2]

    return pl.pallas_call(
        body,
        grid=(_TCG,),
        in_specs=[_rows(bn, 1), _rows(bn, 8), _rows(bn, 8), _rows(bn, 8),
                  _full(1, 8)],
        out_specs=(_rows(bn, 1), _rows(bn, 1)),
        out_shape=(jax.ShapeDtypeStruct((NP, 1), _f32),
                   jax.ShapeDtypeStruct((NP, 1), _f32)),
    )(dinv, u3a, u3b, v3, b34)


def kernel(x, edge_index, W1, b1, W2, b2, W3, b3, W4, b4):
    N, D = x.shape
    H = W1.shape[1]
    E = edge_index.shape[1]

    NP = pl.cdiv(N + 1, CH) * CH
    T = pl.cdiv(E, NS * CH * NBUF) * NBUF

    def _split(frac):
        c0 = min(T - NBUF, max(NBUF, int(round(frac * T / NBUF)) * NBUF))
        return c0, T - c0

    C0a, C1a = _split(0.53)
    C0b, C1b = _split(0.57)
    C0c, C1c = _split(0.60)
    tail = max(C0a - C1a, C0b - C1b, C0c - C1c, 0)
    rows_flat = NS * T + tail
    rows_pt = NP // NS

    pad = jnp.full((rows_flat * CH - E,), N, dtype=edge_index.dtype)
    src_i = jnp.concatenate([edge_index[0], pad]).reshape(rows_flat, CH)
    dst_i = jnp.concatenate([edge_index[1], pad]).reshape(rows_flat, CH)
    x_pad = jnp.pad(x, ((0, NP - N), (0, 0)))
    z32 = jnp.zeros((rows_pt, H), _f32)
    z8 = jnp.zeros((rows_pt, 8), _f32)
    ones8 = jnp.ones((CH, 8), _f32)
    b1r = b1.reshape(1, H)
    b2r = b2.reshape(1, H)
    w3p = jnp.pad(W3, ((0, 0), (0, 7)))
    w4p = jnp.pad(W4, ((0, 0), (1, 6)))
    b34 = jnp.pad(b3.reshape(1, 1), ((0, 0), (0, 7))) + \
        jnp.pad(b4.reshape(1, 1), ((0, 0), (1, 6)))

    p1 = _tc0_call(x_pad, W1, NP, D, H)

    ucnt = _count_call(dst_i, z8, ones8, NP, C0c, C1c)
    cnt_a = ucnt[0, :, 0:1]
    cnt_b = ucnt[1, :, 0:1]

    dinv, v1 = _tc1_call(cnt_a, cnt_b, p1, NP, H)

    u1 = _agg_call(v1, src_i, dst_i, z32, NP, H, C0a, C1a)
    v2 = _tc2_call(dinv, u1[0], u1[1], v1, W2, b1r, NP, H)

    u2 = _agg_call(v2, src_i, dst_i, z32, NP, H, C0a, C1a)
    v3 = _tc3_call(dinv, u2[0], u2[1], v2, w3p, w4p, b2r, NP, H)

    u3 = _agg_call(v3, src_i, dst_i, z8, NP, 8, C0b, C1b)
    mu_f, std_f = _tc4_call(dinv, u3[0], u3[1], v3, b34, NP)

    return (mu_f[:N], std_f[:N])


# speedup vs baseline: 1.1633x; 1.1633x over previous
import functools

import jax
import jax.numpy as jnp
from jax import lax
from jax.experimental import pallas as pl
from jax.experimental.pallas import tpu as pltpu
from jax.experimental.pallas import tpu_sc as plsc

NC = 2
NS = 16
NW = NC * NS
CH = 128
NBUF = 4

_f32 = jnp.float32


def _agg_call(table, ei, zrows, NP, W, C0, C1):
    rows_pt = NP // NS

    @functools.partial(
        pl.kernel,
        out_type=jax.ShapeDtypeStruct((NC, NP, W), _f32),
        mesh=plsc.VectorSubcoreMesh(core_axis_name="c", subcore_axis_name="s"),
        compiler_params=pltpu.CompilerParams(use_tc_tiling_on_sc=False),
        scratch_types=[
            pltpu.VMEM((C0, CH), jnp.int32),
            pltpu.VMEM((C0, CH), jnp.int32),
            pltpu.VMEM((NBUF, CH, W), _f32),
            pltpu.VMEM_SHARED((NP, W), _f32),
            pltpu.VMEM_SHARED((NP, W), _f32),
            pltpu.SemaphoreType.DMA,
            pltpu.SemaphoreType.DMA,
            pltpu.SemaphoreType.DMA,
            pltpu.SemaphoreType.DMA,
            pltpu.SemaphoreType.DMA,
            pltpu.SemaphoreType.DMA,
            pltpu.SemaphoreType.DMA,
            pltpu.SemaphoreType.DMA,
        ],
    )
    def agg(table_h, ei_h, z_h, out_h, src_v, dst_v, rows_v, acc,
            table_s, g0, g1, g2, g3, s0, s1, s2, s3):
        gsem = (g0, g1, g2, g3)
        ssem = (s0, s1, s2, s3)
        cid = lax.axis_index("c")
        sid = lax.axis_index("s")
        row0 = sid * rows_pt
        chunk0 = jnp.where(cid == 0, sid * C0, NS * C0 + sid * C1)
        nch = jnp.where(cid == 0, C0, C1)
        pltpu.sync_copy(z_h, acc.at[pl.ds(row0, rows_pt)])
        pltpu.sync_copy(table_h.at[pl.ds(row0, rows_pt)],
                        table_s.at[pl.ds(row0, rows_pt)])
        pltpu.sync_copy(ei_h.at[0, pl.ds(chunk0, C0)], src_v)
        pltpu.sync_copy(ei_h.at[1, pl.ds(chunk0, C0)], dst_v)
        plsc.subcore_barrier()
        for b in range(2):
            pltpu.async_copy(table_s.at[src_v.at[b]], rows_v.at[b], gsem[b])

        def step(g, carry):
            for b in range(NBUF):
                c = g * NBUF + b
                pltpu.make_async_copy(
                    table_s.at[src_v.at[c]], rows_v.at[b], gsem[b]).wait()
                pltpu.async_copy(rows_v.at[b], acc.at[dst_v.at[c]], ssem[b],
                                 add=True)
                j = c + 2
                bj = (b + 2) % NBUF

                @pl.when(j < nch)
                def _fire():
                    @pl.when(j >= NBUF)
                    def _drain():
                        pltpu.make_async_copy(
                            rows_v.at[bj], acc.at[dst_v.at[j - NBUF]],
                            ssem[bj]).wait()

                    pltpu.async_copy(
                        table_s.at[src_v.at[j]], rows_v.at[bj], gsem[bj])
            return carry

        lax.fori_loop(0, nch // NBUF, step, 0)
        for b in range(NBUF):
            c = nch - NBUF + b
            pltpu.make_async_copy(
                rows_v.at[b], acc.at[dst_v.at[c]], ssem[b]).wait()
        plsc.subcore_barrier()
        pltpu.sync_copy(acc.at[pl.ds(row0, rows_pt)],
                        out_h.at[cid, pl.ds(row0, rows_pt)])

    return agg(table, ei, zrows)


def _count_call(ei, zrows, ones_rows, NP, C0, C1):
    rows_pt = NP // NS

    @functools.partial(
        pl.kernel,
        out_type=jax.ShapeDtypeStruct((NC, NP, 8), _f32),
        mesh=plsc.VectorSubcoreMesh(core_axis_name="c", subcore_axis_name="s"),
        compiler_params=pltpu.CompilerParams(use_tc_tiling_on_sc=False),
        scratch_types=[
            pltpu.VMEM((C0, CH), jnp.int32),
            pltpu.VMEM((CH, 8), _f32),
            pltpu.VMEM_SHARED((NP, 8), _f32),
            pltpu.SemaphoreType.DMA,
            pltpu.SemaphoreType.DMA,
            pltpu.SemaphoreType.DMA,
            pltpu.SemaphoreType.DMA,
        ],
    )
    def cnt(ei_h, z_h, ones_h, out_h, dst_v, ones_v, acc, s0, s1, s2, s3):
        ssem = (s0, s1, s2, s3)
        cid = lax.axis_index("c")
        sid = lax.axis_index("s")
        row0 = sid * rows_pt
        chunk0 = jnp.where(cid == 0, sid * C0, NS * C0 + sid * C1)
        nch = jnp.where(cid == 0, C0, C1)
        pltpu.sync_copy(z_h, acc.at[pl.ds(row0, rows_pt)])
        pltpu.sync_copy(ones_h, ones_v)
        pltpu.sync_copy(ei_h.at[1, pl.ds(chunk0, C0)], dst_v)
        plsc.subcore_barrier()

        def step(g, carry):
            for b in range(NBUF):
                c = g * NBUF + b

                @pl.when(c >= NBUF)
                def _drain():
                    pltpu.make_async_copy(
                        ones_v, acc.at[dst_v.at[c - NBUF]], ssem[b]).wait()

                pltpu.async_copy(ones_v, acc.at[dst_v.at[c]], ssem[b],
                                 add=True)
            return carry

        lax.fori_loop(0, nch // NBUF, step, 0)
        for b in range(NBUF):
            c = nch - NBUF + b
            pltpu.make_async_copy(ones_v, acc.at[dst_v.at[c]], ssem[b]).wait()
        plsc.subcore_barrier()
        pltpu.sync_copy(acc.at[pl.ds(row0, rows_pt)],
                        out_h.at[cid, pl.ds(row0, rows_pt)])

    return cnt(ei, zrows, ones_rows)


_TCG = 8


def _rows(bn, w):
    return pl.BlockSpec((bn, w), lambda i: (i, 0))


def _full(s0, s1):
    return pl.BlockSpec((s0, s1), lambda i: (0, 0))


def _tc0_call(x_pad, W1, NP, D, H):
    bn = NP // _TCG

    def body(xr, w1, p_o):
        p_o[...] = jnp.dot(xr[...], w1[...], preferred_element_type=_f32)

    return pl.pallas_call(
        body,
        grid=(_TCG,),
        in_specs=[_rows(bn, D), _full(D, H)],
        out_specs=_rows(bn, H),
        out_shape=jax.ShapeDtypeStruct((NP, H), _f32),
    )(x_pad, W1)


def _part(bn, w, c):
    return pl.BlockSpec((1, bn, w), lambda i, _c=c: (_c, i, 0))


def _tc1_call(ucnt, p1, NP, H):
    bn = NP // _TCG

    def body(ca, cb, p, dinv_o, v1_o):
        dinv = lax.rsqrt(ca[0, :, 0:1] + cb[0, :, 0:1] + 1.0)
        dinv_o[...] = dinv
        v1_o[...] = p[...] * dinv

    return pl.pallas_call(
        body,
        grid=(_TCG,),
        in_specs=[_part(bn, 8, 0), _part(bn, 8, 1), _rows(bn, H)],
        out_specs=(_rows(bn, 1), _rows(bn, H)),
        out_shape=(jax.ShapeDtypeStruct((NP, 1), _f32),
                   jax.ShapeDtypeStruct((NP, H), _f32)),
    )(ucnt, ucnt, p1)


def _tc2_call(dinv, u1, v1, W2, b1, NP, H):
    bn = NP // _TCG

    def body(di, ua, ub, v, w2, b, v2_o):
        h = di[...] * (ua[0] + ub[0] + v[...]) + b[...]
        v2_o[...] = di[...] * jnp.dot(h, w2[...], preferred_element_type=_f32)

    return pl.pallas_call(
        body,
        grid=(_TCG,),
        in_specs=[_rows(bn, 1), _part(bn, H, 0), _part(bn, H, 1),
                  _rows(bn, H), _full(H, H), _full(1, H)],
        out_specs=_rows(bn, H),
        out_shape=jax.ShapeDtypeStruct((NP, H), _f32),
    )(dinv, u1, u1, v1, W2, b1)


def _tc3_call(dinv, u2, v2, w3p, w4p, b2, NP, H):
    bn = NP // _TCG

    def body(di, ua, ub, v, w3, w4, b, v3_o):
        h2 = di[...] * (ua[0] + ub[0] + v[...]) + b[...]
        r = (jnp.dot(jnp.maximum(h2, 0.0), w3[...], preferred_element_type=_f32)
             + jnp.dot(h2, w4[...], preferred_element_type=_f32))
        v3_o[...] = di[...] * r

    return pl.pallas_call(
        body,
        grid=(_TCG,),
        in_specs=[_rows(bn, 1), _part(bn, H, 0), _part(bn, H, 1),
                  _rows(bn, H), _full(H, 8), _full(H, 8), _full(1, H)],
        out_specs=_rows(bn, 8),
        out_shape=jax.ShapeDtypeStruct((NP, 8), _f32),
    )(dinv, u2, u2, v2, w3p, w4p, b2)


def _tc4_call(dinv, u3, v3, b34, NP):
    bn = NP // _TCG

    def body(di, ua, ub, v, b, mu_o, std_o):
        od = di[...] * (ua[0] + ub[0] + v[...]) + b[...]
        mu_o[...] = od[:, 0:1]
        std_o[...] = od[:, 1:2]

    return pl.pallas_call(
        body,
        grid=(_TCG,),
        in_specs=[_rows(bn, 1), _part(bn, 8, 0), _part(bn, 8, 1),
                  _rows(bn, 8), _full(1, 8)],
        out_specs=(_rows(bn, 1), _rows(bn, 1)),
        out_shape=(jax.ShapeDtypeStruct((NP, 1), _f32),
                   jax.ShapeDtypeStruct((NP, 1), _f32)),
    )(dinv, u3, u3, v3, b34)


def kernel(x, edge_index, W1, b1, W2, b2, W3, b3, W4, b4):
    N, D = x.shape
    H = W1.shape[1]
    E = edge_index.shape[1]

    NP = pl.cdiv(N + 1, CH) * CH
    T = pl.cdiv(E, NS * CH * NBUF) * NBUF

    def _split(frac):
        c0 = min(T - NBUF, max(NBUF, int(round(frac * T / NBUF)) * NBUF))
        return c0, T - c0

    C0a, C1a = _split(0.53)
    C0b, C1b = _split(0.57)
    C0c, C1c = _split(0.60)
    tail = max(C0a - C1a, C0b - C1b, C0c - C1c, 0)
    rows_flat = NS * T + tail
    rows_pt = NP // NS

    ei = jnp.pad(edge_index, ((0, 0), (0, rows_flat * CH - E)),
                 constant_values=N).reshape(2, rows_flat, CH)
    x_pad = jnp.pad(x, ((0, NP - N), (0, 0)))
    z32 = jnp.zeros((rows_pt, H), _f32)
    z8 = jnp.zeros((rows_pt, 8), _f32)
    ones8 = jnp.ones((CH, 8), _f32)
    b1r = b1.reshape(1, H)
    b2r = b2.reshape(1, H)
    w3p = jnp.pad(W3, ((0, 0), (0, 7)))
    w4p = jnp.pad(W4, ((0, 0), (1, 6)))
    b34 = jnp.pad(b3.reshape(1, 1), ((0, 0), (0, 7))) + \
        jnp.pad(b4.reshape(1, 1), ((0, 0), (1, 6)))

    p1 = _tc0_call(x_pad, W1, NP, D, H)

    ucnt = _count_call(ei, z8, ones8, NP, C0c, C1c)

    dinv, v1 = _tc1_call(ucnt, p1, NP, H)

    u1 = _agg_call(v1, ei, z32, NP, H, C0a, C1a)
    v2 = _tc2_call(dinv, u1, v1, W2, b1r, NP, H)

    u2 = _agg_call(v2, ei, z32, NP, H, C0a, C1a)
    v3 = _tc3_call(dinv, u2, v2, w3p, w4p, b2r, NP, H)

    u3 = _agg_call(v3, ei, z8, NP, 8, C0b, C1b)
    mu_f, std_f = _tc4_call(dinv, u3, v3, b34, NP)

    return (mu_f[:N], std_f[:N])
